# baseline (speedup 1.0000x reference)
import functools

import jax
import jax.numpy as jnp
from jax import lax
from jax.experimental import pallas as pl
from jax.experimental.pallas import tpu as pltpu
from jax.experimental.pallas import tpu_sc as plsc

N = 10000
E = 320000
D = 128
H = 8
C = 16
FF = 512
NPAD = 10240
NUM_CORES = 2
NUM_SUBCORES = 16
NW = NUM_CORES * NUM_SUBCORES
SCCH = 1600


def _layernorm(v, g, b):
    mu = jnp.mean(v, axis=-1, keepdims=True)
    var = jnp.mean((v - mu) * (v - mu), axis=-1, keepdims=True)
    return (v - mu) / jnp.sqrt(var + 1e-5) * g + b


def _proj_body(x_ref, wl_ref, bl_ref, wr_ref, br_ref, p_ref,
               hl_ref, hr_ref, gm_ref, qm_ref):
    i = pl.program_id(0)
    x = x_ref[...]
    hl = jnp.dot(x, wl_ref[...], preferred_element_type=jnp.float32,
                 precision=lax.Precision.HIGHEST) + bl_ref[...]
    hr = jnp.dot(x, wr_ref[...], preferred_element_type=jnp.float32,
                 precision=lax.Precision.HIGHEST) + br_ref[...]
    hl_ref[...] = hl
    hr_ref[...] = hr
    gb = jnp.dot(jnp.abs(hl), p_ref[...], preferred_element_type=jnp.float32,
                 precision=lax.Precision.HIGHEST)
    qb = jnp.dot(jnp.abs(hr), p_ref[...], preferred_element_type=jnp.float32,
                 precision=lax.Precision.HIGHEST)
    gmax = jnp.broadcast_to(jnp.max(gb, axis=0, keepdims=True), (8, gb.shape[-1]))
    qmax = jnp.broadcast_to(jnp.max(qb, axis=0, keepdims=True), (8, qb.shape[-1]))

    @pl.when(i == 0)
    def _():
        gm_ref[...] = gmax
        qm_ref[...] = qmax

    @pl.when(i > 0)
    def _():
        gm_ref[...] = jnp.maximum(gm_ref[...], gmax)
        qm_ref[...] = jnp.maximum(qm_ref[...], qmax)


def _make_proj(n, d, blk):
    grid = n // blk
    return pl.pallas_call(
        _proj_body,
        grid=(grid,),
        in_specs=[
            pl.BlockSpec((blk, d), lambda i: (i, 0)),
            pl.BlockSpec((d, d), lambda i: (0, 0)),
            pl.BlockSpec((1, d), lambda i: (0, 0)),
            pl.BlockSpec((d, d), lambda i: (0, 0)),
            pl.BlockSpec((1, d), lambda i: (0, 0)),
            pl.BlockSpec((d, d), lambda i: (0, 0)),
        ],
        out_specs=[
            pl.BlockSpec((blk, d), lambda i: (i, 0)),
            pl.BlockSpec((blk, d), lambda i: (i, 0)),
            pl.BlockSpec((8, d), lambda i: (0, 0)),
            pl.BlockSpec((8, d), lambda i: (0, 0)),
        ],
        out_shape=[
            jax.ShapeDtypeStruct((n, d), jnp.float32),
            jax.ShapeDtypeStruct((n, d), jnp.float32),
            jax.ShapeDtypeStruct((8, d), jnp.float32),
            jax.ShapeDtypeStruct((8, d), jnp.float32),
        ],
    )


def _make_edge(n, npad, e, interpret=False):
    own = npad // NW
    nsc = e // SCCH
    ngr = SCCH // 16
    bs = 32
    cap = SCCH + bs
    mesh = plsc.VectorSubcoreMesh(core_axis_name="c", subcore_axis_name="s",
                                  num_cores=NUM_CORES,
                                  num_subcores=NUM_SUBCORES)

    @functools.partial(
        pl.kernel,
        out_type=[
            jax.ShapeDtypeStruct((npad, D), jnp.float32),
            jax.ShapeDtypeStruct((npad * 16,), jnp.float32),
        ],
        mesh=mesh,
        interpret=interpret,
        compiler_params=pltpu.CompilerParams(needs_layout_passes=False),
        scratch_types=[
            pltpu.VMEM((SCCH,), jnp.int32),
            pltpu.VMEM((SCCH,), jnp.int32),
            pltpu.VMEM((cap,), jnp.int32),
            pltpu.VMEM((cap,), jnp.int32),
            pltpu.VMEM((bs,), jnp.int32),
            pltpu.VMEM((bs, D), jnp.float32),
            pltpu.VMEM((own + 1, D), jnp.float32),
            pltpu.VMEM((bs * 16,), jnp.float32),
            pltpu.VMEM((own + 1, D), jnp.float32),
            pltpu.VMEM(((own + 1) * 16,), jnp.float32),
            pltpu.VMEM((H * C,), jnp.float32),
            pltpu.VMEM((16,), jnp.float32),
            pltpu.SemaphoreType.DMA,
            pltpu.SemaphoreType.DMA,
        ],
    )
    def pe(hl_hbm, hr_hbm, src_hbm, dst_hbm, att_hbm, m_hbm,
           out_hbm, den_hbm,
           srcs, dsts, srcc, dstc, srcg, hl16, hro, exb,
           acc, den, attb, m16, sem_a, sem_b):
        cid = lax.axis_index("c")
        sid = lax.axis_index("s")
        wid = cid * NUM_SUBCORES + sid
        lo = wid * own
        iota16 = lax.iota(jnp.int32, 16)
        zero16 = jnp.zeros((16,), jnp.float32)
        izero16 = jnp.zeros((16,), jnp.int32)

        pltpu.sync_copy(att_hbm, attb)
        pltpu.sync_copy(m_hbm, m16)
        pltpu.sync_copy(hr_hbm.at[pl.ds(lo, own)], hro.at[pl.ds(0, own)])

        def _z(r, carry):
            for j in range(D // 16):
                acc[r, pl.ds(j * 16, 16)] = zero16
            den[pl.ds(r * 16, 16)] = zero16
            return carry
        lax.fori_loop(0, own + 1, _z, 0)
        for j in range(D // 16):
            hro[own, pl.ds(j * 16, 16)] = zero16

        def _zi(r, carry):
            srcc[pl.ds(r * 16, 16)] = izero16
            dstc[pl.ds(r * 16, 16)] = izero16
            return carry
        lax.fori_loop(0, cap // 16, _zi, 0)

        def scan_chunk(s, carry):
            sb = s * SCCH
            cps = pltpu.async_copy(src_hbm.at[pl.ds(sb, SCCH)], srcs, sem_a)
            cpd = pltpu.async_copy(dst_hbm.at[pl.ds(sb, SCCH)], dsts, sem_b)
            cps.wait()
            cpd.wait()

            def grp(g, off):
                sv = srcs[pl.ds(g * 16, 16)]
                dv = dsts[pl.ds(g * 16, 16)]
                msk = (dv >= lo) & (dv < lo + own)
                plsc.store_compressed(srcc.at[pl.ds(off, 16)], sv, mask=msk)
                plsc.store_compressed(dstc.at[pl.ds(off, 16)], dv, mask=msk)
                cntv = plsc.all_reduce_population_count(msk)
                return off + cntv[0]
            cnt = lax.fori_loop(0, ngr, grp, jnp.int32(0))

            nb = (cnt + bs - 1) // bs
            m16v = m16[...]

            def batch(b, carry2):
                p = b * bs
                for q in range(bs // 16):
                    srcg[pl.ds(q * 16, 16)] = srcc[pl.ds(p + q * 16, 16)]
                cpa = pltpu.async_copy(hl_hbm.at[srcg], hl16, sem_a)
                cpa.wait()
                rem = cnt - p

                for g4 in range(bs // 16):
                    rows = g4 * 16 + iota16
                    dv = dstc[pl.ds(p + g4 * 16, 16)]
                    dloc = jnp.where(g4 * 16 + iota16 < rem, dv - lo, own)
                    for h in range(H):
                        attrow = attb[pl.ds(h * C, 16)]
                        lacc = jnp.zeros((16,), jnp.float32)
                        for c in range(C):
                            cols = jnp.full((16,), h * C + c, jnp.int32)
                            a = plsc.load_gather(hl16, [rows, cols])
                            bq = plsc.load_gather(hro, [dloc, cols])
                            mm = a + bq
                            mm = jnp.maximum(mm, 0.2 * mm)
                            lacc = lacc + mm * attrow[c]
                        exv = jnp.exp(lacc - m16v[h])
                        plsc.store_scatter(exb, [rows * 16 + h], exv)
                    for j in range(16):
                        dj = dloc[j]
                        ej = g4 * 16 + j
                        exv = exb[pl.ds(ej * 16, 16)]
                        den[pl.ds(dj * 16, 16)] = den[pl.ds(dj * 16, 16)] + exv
                        for h in range(H):
                            acc[dj, pl.ds(h * C, C)] = (
                                acc[dj, pl.ds(h * C, C)]
                                + hl16[ej, pl.ds(h * C, C)] * exv[h])
                return carry2
            lax.fori_loop(0, nb, batch, 0)
            return carry
        lax.fori_loop(0, nsc, scan_chunk, 0)

        pltpu.sync_copy(acc.at[pl.ds(0, own)], out_hbm.at[pl.ds(lo, own)])
        pltpu.sync_copy(den.at[pl.ds(0, own * 16)], den_hbm.at[pl.ds(lo * 16, own * 16)])

    return pe


def _make_p1b(npad, interpret=False):
    rows_pw = npad // NW
    mesh = plsc.VectorSubcoreMesh(core_axis_name="c", subcore_axis_name="s",
                                  num_cores=NUM_CORES,
                                  num_subcores=NUM_SUBCORES)

    @functools.partial(
        pl.kernel,
        out_type=jax.ShapeDtypeStruct((npad, D), jnp.float32),
        mesh=mesh,
        interpret=interpret,
        compiler_params=pltpu.CompilerParams(needs_layout_passes=False),
        scratch_types=[
            pltpu.VMEM((rows_pw * 16,), jnp.float32),
            pltpu.VMEM((rows_pw, D), jnp.float32),
        ],
    )
    def p1b(den_hbm, rden_hbm, d00, dx):
        cid = lax.axis_index("c")
        sid = lax.axis_index("s")
        wid = cid * NUM_SUBCORES + sid
        r0 = wid * rows_pw
        pltpu.sync_copy(den_hbm.at[pl.ds(r0 * 16, rows_pw * 16)], d00)

        def body(r, carry):
            v = d00[pl.ds(r * 16, 16)]
            rv = 1.0 / (v + 1e-16)
            for h in range(H):
                dx[r, pl.ds(h * C, C)] = jnp.full((16,), rv[h], jnp.float32)
            return carry
        lax.fori_loop(0, rows_pw, body, 0)
        pltpu.sync_copy(dx, rden_hbm.at[pl.ds(r0, rows_pw)])

    return p1b


def _ffn_body(o0_ref, rd_ref, gb_ref, g1_ref, b1n_ref, w1_ref, bb1_ref,
              w2_ref, bb2_ref, g2_ref, b2n_ref, z_ref):
    o = o0_ref[...] * rd_ref[...] + gb_ref[...]
    hn = _layernorm(o, g1_ref[...], b1n_ref[...])
    f = jnp.dot(hn, w1_ref[...], preferred_element_type=jnp.float32,
                precision=lax.Precision.HIGHEST) + bb1_ref[...]
    f = f * (1.0 / (1.0 + jnp.exp(-f)))
    y = o + jnp.dot(f, w2_ref[...], preferred_element_type=jnp.float32,
                    precision=lax.Precision.HIGHEST) + bb2_ref[...]
    z_ref[...] = _layernorm(y, g2_ref[...], b2n_ref[...])


def _make_ffn(npad, d, ff, blk):
    grid = npad // blk
    return pl.pallas_call(
        _ffn_body,
        grid=(grid,),
        in_specs=[
            pl.BlockSpec((blk, d), lambda i: (i, 0)),
            pl.BlockSpec((blk, d), lambda i: (i, 0)),
            pl.BlockSpec((1, d), lambda i: (0, 0)),
            pl.BlockSpec((1, d), lambda i: (0, 0)),
            pl.BlockSpec((1, d), lambda i: (0, 0)),
            pl.BlockSpec((d, ff), lambda i: (0, 0)),
            pl.BlockSpec((1, ff), lambda i: (0, 0)),
            pl.BlockSpec((ff, d), lambda i: (0, 0)),
            pl.BlockSpec((1, d), lambda i: (0, 0)),
            pl.BlockSpec((1, d), lambda i: (0, 0)),
            pl.BlockSpec((1, d), lambda i: (0, 0)),
        ],
        out_specs=pl.BlockSpec((blk, d), lambda i: (i, 0)),
        out_shape=jax.ShapeDtypeStruct((npad, d), jnp.float32),
    )


def kernel(x, edge_index, W_l, b_l, W_r, b_r, att, gat_bias,
           ln1_g, ln1_b, W1, b1, W2, b2, ln2_g, ln2_b):
    xf = x.reshape(N, D)
    src = edge_index[0]
    dst = edge_index[1]

    absatt = jnp.abs(att).reshape(H * C)
    P = jnp.zeros((D, D), jnp.float32).at[jnp.arange(D), jnp.arange(D) // C].set(absatt)

    proj = _make_proj(N, D, 2000)
    hl, hr, gm, qm = proj(xf, W_l, b_l.reshape(1, D), W_r, b_r.reshape(1, D), P)
    m16 = jnp.pad(gm[0, :H] + qm[0, :H], (0, 16 - H))

    pe = _make_edge(N, NPAD, E)
    outu, den = pe(hl, hr, src, dst, att.reshape(H * C), m16)

    p1b = _make_p1b(NPAD)
    rdenx = p1b(den)

    ffn = _make_ffn(NPAD, D, FF, 2560)
    z = ffn(outu, rdenx, gat_bias.reshape(1, D), ln1_g.reshape(1, D),
            ln1_b.reshape(1, D), W1, b1.reshape(1, FF), W2,
            b2.reshape(1, D), ln2_g.reshape(1, D), ln2_b.reshape(1, D))
    return z[:N].reshape(1, N, D)


# speedup vs baseline: 1.1362x; 1.1362x over previous
import functools

import jax
import jax.numpy as jnp
from jax import lax
from jax.experimental import pallas as pl
from jax.experimental.pallas import tpu as pltpu
from jax.experimental.pallas import tpu_sc as plsc

N = 10000
E = 320000
D = 128
H = 8
C = 16
FF = 512
NPAD = 10240
NUM_CORES = 2
NUM_SUBCORES = 16
NW = NUM_CORES * NUM_SUBCORES
SCCH = 1600


def _layernorm(v, g, b):
    mu = jnp.mean(v, axis=-1, keepdims=True)
    var = jnp.mean((v - mu) * (v - mu), axis=-1, keepdims=True)
    return (v - mu) / jnp.sqrt(var + 1e-5) * g + b


def _proj_body(x_ref, wl_ref, bl_ref, wr_ref, br_ref, p_ref,
               hl_ref, hr_ref, gm_ref, qm_ref):
    i = pl.program_id(0)
    x = x_ref[...]
    hl = jnp.dot(x, wl_ref[...], preferred_element_type=jnp.float32,
                 precision=lax.Precision.HIGHEST) + bl_ref[...]
    hr = jnp.dot(x, wr_ref[...], preferred_element_type=jnp.float32,
                 precision=lax.Precision.HIGHEST) + br_ref[...]
    hl_ref[...] = hl
    hr_ref[...] = hr
    gb = jnp.dot(jnp.abs(hl), p_ref[...], preferred_element_type=jnp.float32,
                 precision=lax.Precision.HIGHEST)
    qb = jnp.dot(jnp.abs(hr), p_ref[...], preferred_element_type=jnp.float32,
                 precision=lax.Precision.HIGHEST)
    gmax = jnp.broadcast_to(jnp.max(gb, axis=0, keepdims=True), (8, gb.shape[-1]))
    qmax = jnp.broadcast_to(jnp.max(qb, axis=0, keepdims=True), (8, qb.shape[-1]))

    @pl.when(i == 0)
    def _():
        gm_ref[...] = gmax
        qm_ref[...] = qmax

    @pl.when(i > 0)
    def _():
        gm_ref[...] = jnp.maximum(gm_ref[...], gmax)
        qm_ref[...] = jnp.maximum(qm_ref[...], qmax)


def _make_proj(n, d, blk):
    grid = n // blk
    return pl.pallas_call(
        _proj_body,
        grid=(grid,),
        in_specs=[
            pl.BlockSpec((blk, d), lambda i: (i, 0)),
            pl.BlockSpec((d, d), lambda i: (0, 0)),
            pl.BlockSpec((1, d), lambda i: (0, 0)),
            pl.BlockSpec((d, d), lambda i: (0, 0)),
            pl.BlockSpec((1, d), lambda i: (0, 0)),
            pl.BlockSpec((d, d), lambda i: (0, 0)),
        ],
        out_specs=[
            pl.BlockSpec((blk, d), lambda i: (i, 0)),
            pl.BlockSpec((blk, d), lambda i: (i, 0)),
            pl.BlockSpec((8, d), lambda i: (0, 0)),
            pl.BlockSpec((8, d), lambda i: (0, 0)),
        ],
        out_shape=[
            jax.ShapeDtypeStruct((n, d), jnp.float32),
            jax.ShapeDtypeStruct((n, d), jnp.float32),
            jax.ShapeDtypeStruct((8, d), jnp.float32),
            jax.ShapeDtypeStruct((8, d), jnp.float32),
        ],
    )


def _make_edge(n, npad, e):
    own = npad // NW
    nsc = e // SCCH
    ngr = SCCH // 16
    cap = SCCH + 16
    mesh = plsc.VectorSubcoreMesh(core_axis_name="c", subcore_axis_name="s",
                                  num_cores=NUM_CORES,
                                  num_subcores=NUM_SUBCORES)

    @functools.partial(
        pl.kernel,
        out_type=[
            jax.ShapeDtypeStruct((npad, D), jnp.float32),
            jax.ShapeDtypeStruct((npad, 16), jnp.float32),
        ],
        mesh=mesh,
        compiler_params=pltpu.CompilerParams(needs_layout_passes=False),
        scratch_types=[
            pltpu.VMEM((SCCH,), jnp.int32),
            pltpu.VMEM((SCCH,), jnp.int32),
            pltpu.VMEM((cap,), jnp.int32),
            pltpu.VMEM((cap,), jnp.int32),
            pltpu.VMEM((16,), jnp.int32),
            pltpu.VMEM((16,), jnp.int32),
            pltpu.VMEM((16, D), jnp.float32),
            pltpu.VMEM((16, D), jnp.float32),
            pltpu.VMEM((16, 16), jnp.float32),
            pltpu.VMEM((own + 1, D), jnp.float32),
            pltpu.VMEM((own + 1, 16), jnp.float32),
            pltpu.VMEM((H, C), jnp.float32),
            pltpu.VMEM((16,), jnp.float32),
            pltpu.SemaphoreType.DMA,
            pltpu.SemaphoreType.DMA,
        ],
    )
    def pe(hl_hbm, hr_hbm, src_hbm, dst_hbm, att_hbm, m_hbm,
           out_hbm, den_hbm,
           srcs, dsts, srcc, dstc, srcg, dstg, hl16, hr16, exb,
           acc, den, attb, m16, sem_a, sem_b):
        cid = lax.axis_index("c")
        sid = lax.axis_index("s")
        wid = cid * NUM_SUBCORES + sid
        lo = wid * own
        iota16 = lax.iota(jnp.int32, 16)
        zero16 = jnp.zeros((16,), jnp.float32)
        izero16 = jnp.zeros((16,), jnp.int32)

        pltpu.sync_copy(att_hbm, attb)
        pltpu.sync_copy(m_hbm, m16)

        def _z(r, carry):
            for j in range(D // 16):
                acc[r, pl.ds(j * 16, 16)] = zero16
            den[r] = zero16
            return carry
        lax.fori_loop(0, own + 1, _z, 0)

        def _zi(r, carry):
            srcc[pl.ds(r * 16, 16)] = izero16
            dstc[pl.ds(r * 16, 16)] = izero16
            return carry
        lax.fori_loop(0, cap // 16, _zi, 0)

        def scan_chunk(s, carry):
            sb = s * SCCH
            pltpu.sync_copy(src_hbm.at[pl.ds(sb, SCCH)], srcs)
            pltpu.sync_copy(dst_hbm.at[pl.ds(sb, SCCH)], dsts)

            def grp(g, off):
                sv = srcs[pl.ds(g * 16, 16)]
                dv = dsts[pl.ds(g * 16, 16)]
                msk = (dv >= lo) & (dv < lo + own)
                plsc.store_compressed(srcc.at[pl.ds(off, 16)], sv, mask=msk)
                plsc.store_compressed(dstc.at[pl.ds(off, 16)], dv, mask=msk)
                cntv = plsc.all_reduce_population_count(msk)
                return off + cntv[0]
            cnt = lax.fori_loop(0, ngr, grp, jnp.int32(0))

            nb = (cnt + 15) // 16
            m16v = m16[...]

            def batch(b, carry2):
                p = b * 16
                srcg[...] = srcc[pl.ds(p, 16)]
                dstg[...] = dstc[pl.ds(p, 16)]
                cpa = pltpu.async_copy(hl_hbm.at[srcg], hl16, sem_a)
                cpb = pltpu.async_copy(hr_hbm.at[dstg], hr16, sem_b)
                cpa.wait()
                cpb.wait()
                for h in range(H):
                    attrow = attb[h]
                    lacc = jnp.zeros((16,), jnp.float32)
                    for c in range(C):
                        cols = jnp.full((16,), h * C + c, jnp.int32)
                        a = plsc.load_gather(hl16, [iota16, cols])
                        bq = plsc.load_gather(hr16, [iota16, cols])
                        mm = a + bq
                        mm = jnp.maximum(mm, 0.2 * mm)
                        lacc = lacc + mm * attrow[c]
                    exv = jnp.exp(lacc - m16v[h])
                    plsc.store_scatter(
                        exb, [iota16, jnp.full((16,), h, jnp.int32)], exv)
                dv = dstg[...]
                rem = cnt - p
                dloc = jnp.where(iota16 < rem, dv - lo, own)
                for j in range(16):
                    dj = dloc[j]
                    exv = exb[j]
                    den[dj] = den[dj] + exv
                    for h in range(H):
                        acc[dj, pl.ds(h * C, C)] = (
                            acc[dj, pl.ds(h * C, C)]
                            + hl16[j, pl.ds(h * C, C)] * exv[h])
                return carry2
            lax.fori_loop(0, nb, batch, 0)
            return carry
        lax.fori_loop(0, nsc, scan_chunk, 0)

        pltpu.sync_copy(acc.at[pl.ds(0, own)], out_hbm.at[pl.ds(lo, own)])
        pltpu.sync_copy(den.at[pl.ds(0, own)], den_hbm.at[pl.ds(lo, own)])

    return pe


def _make_p1b(npad):
    rows_pw = npad // NW
    mesh = plsc.VectorSubcoreMesh(core_axis_name="c", subcore_axis_name="s",
                                  num_cores=NUM_CORES,
                                  num_subcores=NUM_SUBCORES)

    @functools.partial(
        pl.kernel,
        out_type=jax.ShapeDtypeStruct((npad, D), jnp.float32),
        mesh=mesh,
        compiler_params=pltpu.CompilerParams(needs_layout_passes=False),
        scratch_types=[
            pltpu.VMEM((rows_pw, 16), jnp.float32),
            pltpu.VMEM((rows_pw, D), jnp.float32),
        ],
    )
    def p1b(den_hbm, rden_hbm, d0, dx):
        cid = lax.axis_index("c")
        sid = lax.axis_index("s")
        wid = cid * NUM_SUBCORES + sid
        r0 = wid * rows_pw
        pltpu.sync_copy(den_hbm.at[pl.ds(r0, rows_pw)], d0)

        def body(r, carry):
            v = d0[r]
            rv = 1.0 / (v + 1e-16)
            for h in range(H):
                dx[r, pl.ds(h * C, C)] = jnp.full((16,), rv[h], jnp.float32)
            return carry
        lax.fori_loop(0, rows_pw, body, 0)
        pltpu.sync_copy(dx, rden_hbm.at[pl.ds(r0, rows_pw)])

    return p1b


def _ffn_body(o0_ref, rd_ref, gb_ref, g1_ref, b1n_ref, w1_ref, bb1_ref,
              w2_ref, bb2_ref, g2_ref, b2n_ref, z_ref):
    o = o0_ref[...] * rd_ref[...] + gb_ref[...]
    hn = _layernorm(o, g1_ref[...], b1n_ref[...])
    f = jnp.dot(hn, w1_ref[...], preferred_element_type=jnp.float32,
                precision=lax.Precision.HIGHEST) + bb1_ref[...]
    f = f * (1.0 / (1.0 + jnp.exp(-f)))
    y = o + jnp.dot(f, w2_ref[...], preferred_element_type=jnp.float32,
                    precision=lax.Precision.HIGHEST) + bb2_ref[...]
    z_ref[...] = _layernorm(y, g2_ref[...], b2n_ref[...])


def _make_ffn(npad, d, ff, blk):
    grid = npad // blk
    return pl.pallas_call(
        _ffn_body,
        grid=(grid,),
        in_specs=[
            pl.BlockSpec((blk, d), lambda i: (i, 0)),
            pl.BlockSpec((blk, d), lambda i: (i, 0)),
            pl.BlockSpec((1, d), lambda i: (0, 0)),
            pl.BlockSpec((1, d), lambda i: (0, 0)),
            pl.BlockSpec((1, d), lambda i: (0, 0)),
            pl.BlockSpec((d, ff), lambda i: (0, 0)),
            pl.BlockSpec((1, ff), lambda i: (0, 0)),
            pl.BlockSpec((ff, d), lambda i: (0, 0)),
            pl.BlockSpec((1, d), lambda i: (0, 0)),
            pl.BlockSpec((1, d), lambda i: (0, 0)),
            pl.BlockSpec((1, d), lambda i: (0, 0)),
        ],
        out_specs=pl.BlockSpec((blk, d), lambda i: (i, 0)),
        out_shape=jax.ShapeDtypeStruct((npad, d), jnp.float32),
    )


def kernel(x, edge_index, W_l, b_l, W_r, b_r, att, gat_bias,
           ln1_g, ln1_b, W1, b1, W2, b2, ln2_g, ln2_b):
    xf = x.reshape(N, D)
    src = edge_index[0]
    dst = edge_index[1]

    absatt = jnp.abs(att).reshape(H * C)
    P = jnp.zeros((D, D), jnp.float32).at[jnp.arange(D), jnp.arange(D) // C].set(absatt)

    proj = _make_proj(N, D, 2000)
    hl, hr, gm, qm = proj(xf, W_l, b_l.reshape(1, D), W_r, b_r.reshape(1, D), P)
    m16 = jnp.pad(gm[0, :H] + qm[0, :H], (0, 16 - H))

    pe = _make_edge(N, NPAD, E)
    outu, den = pe(hl, hr, src, dst, att, m16)

    p1b = _make_p1b(NPAD)
    rdenx = p1b(den)

    ffn = _make_ffn(NPAD, D, FF, 2560)
    z = ffn(outu, rdenx, gat_bias.reshape(1, D), ln1_g.reshape(1, D),
            ln1_b.reshape(1, D), W1, b1.reshape(1, FF), W2,
            b2.reshape(1, D), ln2_g.reshape(1, D), ln2_b.reshape(1, D))
    return z[:N].reshape(1, N, D)
